# baseline (device time: 61834 ns/iter reference)
import jax
import jax.numpy as jnp
from jax import lax
from jax.experimental import pallas as pl
from jax.experimental.pallas import tpu as pltpu

M_LOC = 4096
N_IN = 2048
N_OUT = 1024
M_HALF = 2048

SIZES = [128] * 14 + [64] * 4
OFFS = [sum(SIZES[:i]) for i in range(len(SIZES))]
C = len(SIZES)

S = 8
SCH = M_HALF // S
LC = 16
LCHUNK = M_LOC // LC


def kernel(x):

    def body(x_ref, out_ref, sstage, send_bf, recv_x, lstage, lbf,
             sload_sems, lload_sems, lcopy_sems, ccopy_sems,
             send_sems_x, recv_sems_x, send_sems_y, recv_sems_y):
        px = lax.axis_index("x")
        py = lax.axis_index("y")

        def send_load(j):
            return pltpu.make_async_copy(
                x_ref.at[pl.ds(py * M_HALF + j * SCH, SCH),
                         pl.ds((1 - px) * N_OUT, N_OUT)],
                sstage.at[j % 2],
                sload_sems.at[j % 2],
            )

        send_load(0).start()

        barrier = pltpu.get_barrier_semaphore()
        pl.semaphore_signal(barrier, inc=1, device_id=(1 - px, py),
                            device_id_type=pl.DeviceIdType.MESH)
        pl.semaphore_signal(barrier, inc=1, device_id=(px, 1 - py),
                            device_id_type=pl.DeviceIdType.MESH)
        pl.semaphore_wait(barrier, 2)

        def rdma_x(c):
            return pltpu.make_async_remote_copy(
                src_ref=send_bf.at[pl.ds(OFFS[c], SIZES[c]), :],
                dst_ref=recv_x.at[pl.ds(OFFS[c], SIZES[c]), :],
                send_sem=send_sems_x.at[c],
                recv_sem=recv_sems_x.at[c],
                device_id=(1 - px, py),
                device_id_type=pl.DeviceIdType.MESH,
            )

        chunks_of = [
            [c for c in range(C)
             if j * SCH <= OFFS[c] and OFFS[c] + SIZES[c] <= (j + 1) * SCH]
            for j in range(S)
        ]

        for j in range(S):
            if j + 1 < S:
                send_load(j + 1).start()
            send_load(j).wait()
            send_bf[pl.ds(j * SCH, SCH), :] = (
                sstage[j % 2].astype(jnp.bfloat16))
            for c in chunks_of[j]:
                rdma_x(c).start()

        def local_load(c):
            return pltpu.make_async_copy(
                x_ref.at[pl.ds(c * LCHUNK, LCHUNK),
                         pl.ds(px * N_OUT, N_OUT)],
                lstage.at[c % 2],
                lload_sems.at[c % 2],
            )

        def local_copy_out(c, slot):
            return pltpu.make_async_copy(
                lbf.at[slot],
                out_ref.at[pl.ds(px * M_LOC + c * LCHUNK, LCHUNK), :],
                lcopy_sems.at[slot],
            )

        local_load(0).start()

        def rdma_y(c):
            recv_rows = pl.ds((1 - px) * M_LOC + py * M_HALF + OFFS[c],
                              SIZES[c])
            return pltpu.make_async_remote_copy(
                src_ref=recv_x.at[pl.ds(OFFS[c], SIZES[c]), :],
                dst_ref=out_ref.at[recv_rows, :],
                send_sem=send_sems_y.at[c],
                recv_sem=recv_sems_y.at[c],
                device_id=(px, 1 - py),
                device_id_type=pl.DeviceIdType.MESH,
            )

        def chunk_copy_out(c):
            return pltpu.make_async_copy(
                recv_x.at[pl.ds(OFFS[c], SIZES[c]), :],
                out_ref.at[
                    pl.ds((1 - px) * M_LOC + py * M_HALF + OFFS[c],
                          SIZES[c]), :],
                ccopy_sems.at[c],
            )

        for c in range(C):
            lc = c if c < LC else None
            if lc is not None and lc + 1 < LC:
                local_load(lc + 1).start()
            rdma_x(c).wait_recv()
            rdma_y(c).start()
            chunk_copy_out(c).start()
            if lc is not None:
                slot = lc % 2
                if lc >= 2:
                    local_copy_out(lc - 2, slot).wait()
                local_load(lc).wait()
                lbf[slot] = lstage[slot].astype(jnp.bfloat16)
                local_copy_out(lc, slot).start()

        for c in range(C):
            rdma_y(c).wait()
            rdma_x(c).wait_send()
            chunk_copy_out(c).wait()
        local_copy_out(LC - 2, (LC - 2) % 2).wait()
        local_copy_out(LC - 1, (LC - 1) % 2).wait()

    return pl.pallas_call(
        body,
        out_shape=jax.ShapeDtypeStruct((2 * M_LOC, N_OUT), jnp.bfloat16),
        in_specs=[pl.BlockSpec(memory_space=pl.ANY)],
        out_specs=pl.BlockSpec(memory_space=pl.ANY),
        scratch_shapes=[
            pltpu.VMEM((2, SCH, N_OUT), jnp.float32),
            pltpu.VMEM((M_HALF, N_OUT), jnp.bfloat16),
            pltpu.VMEM((M_HALF, N_OUT), jnp.bfloat16),
            pltpu.VMEM((2, LCHUNK, N_OUT), jnp.float32),
            pltpu.VMEM((2, LCHUNK, N_OUT), jnp.bfloat16),
            pltpu.SemaphoreType.DMA((2,)),
            pltpu.SemaphoreType.DMA((2,)),
            pltpu.SemaphoreType.DMA((2,)),
            pltpu.SemaphoreType.DMA((C,)),
            pltpu.SemaphoreType.DMA((C,)),
            pltpu.SemaphoreType.DMA((C,)),
            pltpu.SemaphoreType.DMA((C,)),
            pltpu.SemaphoreType.DMA((C,)),
        ],
        compiler_params=pltpu.CompilerParams(collective_id=0),
    )(x)
